# baseline (speedup 1.0000x reference)
import functools

import jax
import jax.numpy as jnp
from jax import lax
from jax.experimental import pallas as pl
from jax.experimental.pallas import tpu as pltpu
from jax.experimental.pallas import tpu_sc as plsc

N = 10000
E = 160000
D = 128
N_PAD = 10240
E_PAD = 163840
NC, NS = 2, 16
NW = NC * NS
CH = 128
NCHUNK = E_PAD // (NW * CH)
CH_S = 128
NCHUNK_S = E_PAD // (NW * CH_S)
ROWS_PER_TILE = N_PAD // NS
TRASH = N
DW = D // 2

_f32 = jnp.float32


def _proj_body(ih_ref, h_ref, lh_ref, w1_ref, b1_ref, w2_ref, b2_ref,
               w3_ref, b3_ref, p1_ref, p2_ref, p3_ref):
    s = jnp.concatenate([ih_ref[...], h_ref[...], lh_ref[...]], axis=-1)
    p1_ref[...] = jnp.dot(s, w1_ref[...], preferred_element_type=_f32) + b1_ref[...]
    p2_ref[...] = jnp.dot(s, w2_ref[...], preferred_element_type=_f32) + b2_ref[...]
    p3_ref[...] = jnp.dot(s, w3_ref[...], preferred_element_type=_f32) + b3_ref[...]


def _node_proj(ih, h, lh, W_m1, b_m1, W_m2, b_m2, W_o1, b_o1):
    blk = 400
    grid = (N // blk,)
    full = lambda shape: pl.BlockSpec(shape, lambda i: (0, 0))
    rowb = pl.BlockSpec((blk, D), lambda i: (i, 0))

    return pl.pallas_call(
        _proj_body,
        grid=grid,
        in_specs=[
            rowb, rowb, rowb,
            full((3 * D, D)), full((1, D)),
            full((3 * D, D)), full((1, D)),
            full((3 * D, D)), full((1, D)),
        ],
        out_specs=[rowb, rowb, rowb],
        out_shape=[jax.ShapeDtypeStruct((N, D), _f32)] * 3,
    )(ih, h, lh, W_m1, b_m1.reshape(1, D), W_m2, b_m2.reshape(1, D),
      W_o1, b_o1.reshape(1, D))


_NB = 2


def _gather_pipeline(p1_hbm, p2_hbm, srcg_hbm, dstg_hbm, pre_hbm,
                     sidx, didx, bufA, bufB, semG, semS, ebase, nchunk):
    pltpu.sync_copy(srcg_hbm.at[pl.ds(ebase, nchunk * CH)],
                    sidx.at[pl.ds(0, nchunk * CH)])
    pltpu.sync_copy(dstg_hbm.at[pl.ds(ebase, nchunk * CH)],
                    didx.at[pl.ds(0, nchunk * CH)])

    def issue(kk, b):
        pltpu.async_copy(p1_hbm.at[sidx.at[pl.ds(kk * CH, CH)]], bufA[b], semG[b])
        pltpu.async_copy(p2_hbm.at[didx.at[pl.ds(kk * CH, CH)]], bufB[b], semG[b])

    for b in range(_NB - 1):
        issue(b, b)

    @pl.loop(0, nchunk, step=_NB)
    def _group(k):
        for b in range(_NB):
            kk = k + b
            o = (b + _NB - 1) % _NB

            @pl.when(kk + _NB - 1 < nchunk)
            def _():
                @pl.when(kk >= 1)
                def _():
                    pltpu.make_async_copy(
                        bufA[o], pre_hbm.at[pl.ds(ebase, CH)], semS[o]).wait()
                issue(kk + _NB - 1, o)

            pltpu.make_async_copy(p1_hbm.at[sidx.at[pl.ds(kk * CH, CH)]],
                                  bufA[b], semG[b]).wait()
            pltpu.make_async_copy(p2_hbm.at[didx.at[pl.ds(kk * CH, CH)]],
                                  bufB[b], semG[b]).wait()

            @pl.loop(0, CH, unroll=4)
            def _row(r):
                for j in range(D // 16):
                    sl = (r, pl.ds(j * 16, 16))
                    bufA[b][sl] = bufA[b][sl] + bufB[b][sl]

            pltpu.async_copy(bufA[b], pre_hbm.at[pl.ds(ebase + kk * CH, CH)],
                             semS[b])

    for b in range(_NB):
        pltpu.make_async_copy(bufA[b], pre_hbm.at[pl.ds(ebase, CH)],
                              semS[b]).wait()


PAIR_CHUNKS = 2 * NCHUNK
K0 = 32
K1 = PAIR_CHUNKS - K0


def _gather_body(p1_hbm, p2_hbm, srcg_hbm, dstg_hbm, pre_hbm,
                 sidx, didx, *bufs_and_sems):
    bufA = bufs_and_sems[0:_NB]
    bufB = bufs_and_sems[_NB:2 * _NB]
    semG = bufs_and_sems[2 * _NB:3 * _NB]
    semS = bufs_and_sems[3 * _NB:4 * _NB]
    c = lax.axis_index("c")
    s = lax.axis_index("s")
    pair_base = s * PAIR_CHUNKS * CH

    @pl.when(c == 0)
    def _():
        _gather_pipeline(p1_hbm, p2_hbm, srcg_hbm, dstg_hbm, pre_hbm,
                         sidx, didx, bufA, bufB, semG, semS,
                         pair_base, K0)

    @pl.when(c == 1)
    def _():
        _gather_pipeline(p1_hbm, p2_hbm, srcg_hbm, dstg_hbm, pre_hbm,
                         sidx, didx, bufA, bufB, semG, semS,
                         pair_base + K0 * CH, K1)


def _edge_gather(P1, P2, srcg, dstg):
    mesh = plsc.VectorSubcoreMesh(core_axis_name="c", subcore_axis_name="s")
    run = pl.kernel(
        _gather_body,
        out_type=jax.ShapeDtypeStruct((E_PAD, D), _f32),
        mesh=mesh,
        scratch_types=[
            pltpu.VMEM((max(K0, K1) * CH,), jnp.int32),
            pltpu.VMEM((max(K0, K1) * CH,), jnp.int32),
        ]
        + [pltpu.VMEM((CH, D), _f32)] * (2 * _NB)
        + [pltpu.SemaphoreType.DMA] * (2 * _NB),
    )
    return run(P1, P2, srcg, dstg)


def _mlp_body(a_ref, w1_ref, b1_ref, w2_ref, b2_ref, msg_ref):
    h = jnp.maximum(a_ref[...], 0.0).astype(jnp.bfloat16)
    h = jnp.maximum(jnp.dot(h, w1_ref[...], preferred_element_type=_f32)
                    + b1_ref[...], 0.0).astype(jnp.bfloat16)
    msg_ref[...] = jnp.dot(h, w2_ref[...], preferred_element_type=_f32) + b2_ref[...]


def _edge_mlp(pre, W_mlp1, b_mlp1, W_mlp2, b_mlp2):
    blk = 2048
    grid = (E_PAD // blk,)
    full = lambda shape: pl.BlockSpec(shape, lambda i: (0, 0))
    return pl.pallas_call(
        _mlp_body,
        grid=grid,
        in_specs=[
            pl.BlockSpec((blk, D), lambda i: (i, 0)),
            full((D, D)), full((1, D)),
            full((D, D)), full((1, D)),
        ],
        out_specs=pl.BlockSpec((blk, D), lambda i: (i, 0)),
        out_shape=jax.ShapeDtypeStruct((E_PAD, D), _f32),
    )(pre, W_mlp1.astype(jnp.bfloat16), b_mlp1.reshape(1, D),
      W_mlp2.astype(jnp.bfloat16), b_mlp2.reshape(1, D))


def _scatter_body(msg_hbm, dstg_hbm, zeros_hbm, out_hbm,
                  didx, bufM0, bufM1, acc, semL0, semL1):
    c = lax.axis_index("c")
    s = lax.axis_index("s")
    wid = c * NS + s
    rbase = s * ROWS_PER_TILE
    pltpu.sync_copy(zeros_hbm, acc.at[pl.ds(rbase, ROWS_PER_TILE)])
    pltpu.sync_copy(dstg_hbm.at[wid], didx)
    plsc.subcore_barrier()
    ebase = wid * NCHUNK_S * CH_S
    bufM = (bufM0, bufM1)
    semL = (semL0, semL1)

    def issue(kk, b):
        pltpu.async_copy(msg_hbm.at[pl.ds(ebase + kk * CH_S, CH_S)], bufM[b],
                         semL[b])

    issue(0, 0)

    @pl.loop(0, NCHUNK_S, step=2)
    def _pair(k):
        for b in (0, 1):
            kk = k + b
            o = 1 - b

            @pl.when(kk + 1 < NCHUNK_S)
            def _():
                issue(kk + 1, o)

            pltpu.make_async_copy(msg_hbm.at[pl.ds(ebase, CH_S)], bufM[b],
                                  semL[b]).wait()
            pltpu.sync_copy(bufM[b], acc.at[didx.at[kk]], add=True)

    plsc.subcore_barrier()
    pltpu.sync_copy(acc.at[pl.ds(rbase, ROWS_PER_TILE)],
                    out_hbm.at[c, pl.ds(rbase, ROWS_PER_TILE)])


def _edge_scatter(msg, dstg, zrows):
    mesh = plsc.VectorSubcoreMesh(core_axis_name="c", subcore_axis_name="s")
    run = pl.kernel(
        _scatter_body,
        out_type=jax.ShapeDtypeStruct((NC, N_PAD, D), _f32),
        mesh=mesh,
        scratch_types=[
            pltpu.VMEM((NCHUNK_S, CH_S), jnp.int32),
            pltpu.VMEM((CH_S, D), _f32),
            pltpu.VMEM((CH_S, D), _f32),
            pltpu.VMEM_SHARED((N_PAD, D), _f32),
            pltpu.SemaphoreType.DMA,
            pltpu.SemaphoreType.DMA,
        ],
    )
    return run(msg, dstg, zrows)


def _out_body(o3_ref, s0_ref, s1_ref, w_ref, b_ref, out_ref):
    agg = s0_ref[...] + s1_ref[...]
    out_ref[...] = jnp.maximum(
        o3_ref[...] + jnp.dot(agg, w_ref[...], preferred_element_type=_f32)
        + b_ref[...], 0.0)


def _combine(O3, partials, W_o2, b_o2):
    blk = 400
    grid = (N // blk,)
    full = lambda shape: pl.BlockSpec(shape, lambda i: (0, 0))
    return pl.pallas_call(
        _out_body,
        grid=grid,
        in_specs=[
            pl.BlockSpec((blk, D), lambda i: (i, 0)),
            pl.BlockSpec((None, blk, D), lambda i: (0, i, 0)),
            pl.BlockSpec((None, blk, D), lambda i: (1, i, 0)),
            full((D, D)), full((1, D)),
        ],
        out_specs=pl.BlockSpec((blk, D), lambda i: (i, 0)),
        out_shape=jax.ShapeDtypeStruct((N, D), _f32),
    )(O3, partials, partials, W_o2, b_o2.reshape(1, D))


@jax.jit
def kernel(input_hidden, hidden, last_hidden, batch_assignment, edge_index,
           W_m1, b_m1, W_m2, b_m2, W_mlp1, b_mlp1, W_mlp2, b_mlp2,
           W_o1, b_o1, W_o2, b_o2):
    del batch_assignment
    src = edge_index[0].astype(jnp.int32)
    dst = edge_index[1].astype(jnp.int32)
    src_flat = jnp.zeros((E_PAD,), jnp.int32).at[:E].set(src)
    dstg_flat = jnp.zeros((E_PAD,), jnp.int32).at[:E].set(dst)
    dsts_flat = jnp.full((E_PAD,), TRASH, jnp.int32).at[:E].set(dst)
    srcg = src_flat
    dstg = dstg_flat
    dstg_s = dsts_flat.reshape(NW, NCHUNK_S, CH_S)

    zrows = jnp.zeros((ROWS_PER_TILE, D), _f32)

    P1, P2, O3 = _node_proj(input_hidden, hidden, last_hidden,
                            W_m1, b_m1, W_m2, b_m2, W_o1, b_o1)
    pre = _edge_gather(P1, P2, srcg, dstg)
    msg = _edge_mlp(pre, W_mlp1, b_mlp1, W_mlp2, b_mlp2)
    partials = _edge_scatter(msg, dstg_s, zrows)
    return _combine(O3, partials, W_o2, b_o2)


# speedup vs baseline: 1.1050x; 1.1050x over previous
import functools

import jax
import jax.numpy as jnp
from jax import lax
from jax.experimental import pallas as pl
from jax.experimental.pallas import tpu as pltpu
from jax.experimental.pallas import tpu_sc as plsc

N = 10000
E = 160000
D = 128
N_PAD = 10240
E_PAD = 163840
NC, NS = 2, 16
NW = NC * NS
CH = 128
NCHUNK = E_PAD // (NW * CH)
CH_S = 128
NCHUNK_S = E_PAD // (NW * CH_S)
ROWS_PER_TILE = N_PAD // NS
TRASH = N
DW = D // 2

_f32 = jnp.float32


def _proj_body(ih_ref, h_ref, lh_ref, w1_ref, b1_ref, w2_ref, b2_ref,
               w3_ref, b3_ref, p1_ref, p2_ref, p3_ref):
    s = jnp.concatenate([ih_ref[...], h_ref[...], lh_ref[...]], axis=-1)
    p1_ref[...] = jnp.dot(s, w1_ref[...], preferred_element_type=_f32) + b1_ref[...]
    p2_ref[...] = jnp.dot(s, w2_ref[...], preferred_element_type=_f32) + b2_ref[...]
    p3_ref[...] = jnp.dot(s, w3_ref[...], preferred_element_type=_f32) + b3_ref[...]


def _node_proj(ih, h, lh, W_m1, b_m1, W_m2, b_m2, W_o1, b_o1):
    blk = 400
    grid = (N // blk,)
    full = lambda shape: pl.BlockSpec(shape, lambda i: (0, 0))
    rowb = pl.BlockSpec((blk, D), lambda i: (i, 0))

    return pl.pallas_call(
        _proj_body,
        grid=grid,
        in_specs=[
            rowb, rowb, rowb,
            full((3 * D, D)), full((1, D)),
            full((3 * D, D)), full((1, D)),
            full((3 * D, D)), full((1, D)),
        ],
        out_specs=[rowb, rowb, rowb],
        out_shape=[jax.ShapeDtypeStruct((N, D), _f32)] * 3,
    )(ih, h, lh, W_m1, b_m1.reshape(1, D), W_m2, b_m2.reshape(1, D),
      W_o1, b_o1.reshape(1, D))


_NB = 2


def _gather_pipeline(p1_hbm, p2_hbm, srcg_hbm, dstg_hbm, pre_hbm,
                     sidx, didx, bufA, bufB, semG, semS, ebase, nchunk):
    pltpu.sync_copy(srcg_hbm.at[pl.ds(ebase, nchunk * CH)],
                    sidx.at[pl.ds(0, nchunk * CH)])
    pltpu.sync_copy(dstg_hbm.at[pl.ds(ebase, nchunk * CH)],
                    didx.at[pl.ds(0, nchunk * CH)])

    def issue(kk, b):
        pltpu.async_copy(p1_hbm.at[sidx.at[pl.ds(kk * CH, CH)]], bufA[b], semG[b])
        pltpu.async_copy(p2_hbm.at[didx.at[pl.ds(kk * CH, CH)]], bufB[b], semG[b])

    for b in range(_NB - 1):
        issue(b, b)

    @pl.loop(0, nchunk, step=_NB)
    def _group(k):
        for b in range(_NB):
            kk = k + b
            o = (b + _NB - 1) % _NB

            @pl.when(kk + _NB - 1 < nchunk)
            def _():
                @pl.when(kk >= 1)
                def _():
                    pltpu.make_async_copy(
                        bufA[o], pre_hbm.at[pl.ds(ebase, CH)], semS[o]).wait()
                issue(kk + _NB - 1, o)

            pltpu.make_async_copy(p1_hbm.at[sidx.at[pl.ds(kk * CH, CH)]],
                                  bufA[b], semG[b]).wait()
            pltpu.make_async_copy(p2_hbm.at[didx.at[pl.ds(kk * CH, CH)]],
                                  bufB[b], semG[b]).wait()

            @pl.loop(0, CH, unroll=4)
            def _row(r):
                for j in range(D // 16):
                    sl = (r, pl.ds(j * 16, 16))
                    bufA[b][sl] = bufA[b][sl] + bufB[b][sl]

            pltpu.async_copy(bufA[b], pre_hbm.at[pl.ds(ebase + kk * CH, CH)],
                             semS[b])

    for b in range(_NB):
        pltpu.make_async_copy(bufA[b], pre_hbm.at[pl.ds(ebase, CH)],
                              semS[b]).wait()


PAIR_CHUNKS = 2 * NCHUNK
K0 = 48
K1 = PAIR_CHUNKS - K0


def _gather_body(p1_hbm, p2_hbm, srcg_hbm, dstg_hbm, pre_hbm,
                 sidx, didx, *bufs_and_sems):
    bufA = bufs_and_sems[0:_NB]
    bufB = bufs_and_sems[_NB:2 * _NB]
    semG = bufs_and_sems[2 * _NB:3 * _NB]
    semS = bufs_and_sems[3 * _NB:4 * _NB]
    c = lax.axis_index("c")
    s = lax.axis_index("s")
    pair_base = s * PAIR_CHUNKS * CH

    @pl.when(c == 0)
    def _():
        _gather_pipeline(p1_hbm, p2_hbm, srcg_hbm, dstg_hbm, pre_hbm,
                         sidx, didx, bufA, bufB, semG, semS,
                         pair_base, K0)

    @pl.when(c == 1)
    def _():
        _gather_pipeline(p1_hbm, p2_hbm, srcg_hbm, dstg_hbm, pre_hbm,
                         sidx, didx, bufA, bufB, semG, semS,
                         pair_base + K0 * CH, K1)


def _edge_gather(P1, P2, srcg, dstg):
    mesh = plsc.VectorSubcoreMesh(core_axis_name="c", subcore_axis_name="s")
    run = pl.kernel(
        _gather_body,
        out_type=jax.ShapeDtypeStruct((E_PAD, D), _f32),
        mesh=mesh,
        scratch_types=[
            pltpu.VMEM((max(K0, K1) * CH,), jnp.int32),
            pltpu.VMEM((max(K0, K1) * CH,), jnp.int32),
        ]
        + [pltpu.VMEM((CH, D), _f32)] * (2 * _NB)
        + [pltpu.SemaphoreType.DMA] * (2 * _NB),
    )
    return run(P1, P2, srcg, dstg)


def _mlp_body(a_ref, w1_ref, b1_ref, w2_ref, b2_ref, msg_ref):
    h = jnp.maximum(a_ref[...], 0.0).astype(jnp.bfloat16)
    h = jnp.maximum(jnp.dot(h, w1_ref[...], preferred_element_type=_f32)
                    + b1_ref[...], 0.0).astype(jnp.bfloat16)
    msg_ref[...] = jnp.dot(h, w2_ref[...], preferred_element_type=_f32) + b2_ref[...]


def _edge_mlp(pre, W_mlp1, b_mlp1, W_mlp2, b_mlp2):
    blk = 2048
    grid = (E_PAD // blk,)
    full = lambda shape: pl.BlockSpec(shape, lambda i: (0, 0))
    return pl.pallas_call(
        _mlp_body,
        grid=grid,
        in_specs=[
            pl.BlockSpec((blk, D), lambda i: (i, 0)),
            full((D, D)), full((1, D)),
            full((D, D)), full((1, D)),
        ],
        out_specs=pl.BlockSpec((blk, D), lambda i: (i, 0)),
        out_shape=jax.ShapeDtypeStruct((E_PAD, D), _f32),
    )(pre, W_mlp1.astype(jnp.bfloat16), b_mlp1.reshape(1, D),
      W_mlp2.astype(jnp.bfloat16), b_mlp2.reshape(1, D))


def _scatter_body(msg_hbm, dstg_hbm, zeros_hbm, out_hbm,
                  didx, bufM0, bufM1, acc, semL0, semL1):
    c = lax.axis_index("c")
    s = lax.axis_index("s")
    wid = c * NS + s
    rbase = s * ROWS_PER_TILE
    pltpu.sync_copy(zeros_hbm, acc.at[pl.ds(rbase, ROWS_PER_TILE)])
    pltpu.sync_copy(dstg_hbm.at[wid], didx)
    plsc.subcore_barrier()
    ebase = wid * NCHUNK_S * CH_S
    bufM = (bufM0, bufM1)
    semL = (semL0, semL1)

    def issue(kk, b):
        pltpu.async_copy(msg_hbm.at[pl.ds(ebase + kk * CH_S, CH_S)], bufM[b],
                         semL[b])

    issue(0, 0)

    @pl.loop(0, NCHUNK_S, step=2)
    def _pair(k):
        for b in (0, 1):
            kk = k + b
            o = 1 - b

            @pl.when(kk + 1 < NCHUNK_S)
            def _():
                issue(kk + 1, o)

            pltpu.make_async_copy(msg_hbm.at[pl.ds(ebase, CH_S)], bufM[b],
                                  semL[b]).wait()
            pltpu.sync_copy(bufM[b], acc.at[didx.at[kk]], add=True)

    plsc.subcore_barrier()
    pltpu.sync_copy(acc.at[pl.ds(rbase, ROWS_PER_TILE)],
                    out_hbm.at[c, pl.ds(rbase, ROWS_PER_TILE)])


def _edge_scatter(msg, dstg, zrows):
    mesh = plsc.VectorSubcoreMesh(core_axis_name="c", subcore_axis_name="s")
    run = pl.kernel(
        _scatter_body,
        out_type=jax.ShapeDtypeStruct((NC, N_PAD, D), _f32),
        mesh=mesh,
        scratch_types=[
            pltpu.VMEM((NCHUNK_S, CH_S), jnp.int32),
            pltpu.VMEM((CH_S, D), _f32),
            pltpu.VMEM((CH_S, D), _f32),
            pltpu.VMEM_SHARED((N_PAD, D), _f32),
            pltpu.SemaphoreType.DMA,
            pltpu.SemaphoreType.DMA,
        ],
    )
    return run(msg, dstg, zrows)


def _out_body(o3_ref, s0_ref, s1_ref, w_ref, b_ref, out_ref):
    agg = s0_ref[...] + s1_ref[...]
    out_ref[...] = jnp.maximum(
        o3_ref[...] + jnp.dot(agg, w_ref[...], preferred_element_type=_f32)
        + b_ref[...], 0.0)


def _combine(O3, partials, W_o2, b_o2):
    blk = 400
    grid = (N // blk,)
    full = lambda shape: pl.BlockSpec(shape, lambda i: (0, 0))
    return pl.pallas_call(
        _out_body,
        grid=grid,
        in_specs=[
            pl.BlockSpec((blk, D), lambda i: (i, 0)),
            pl.BlockSpec((None, blk, D), lambda i: (0, i, 0)),
            pl.BlockSpec((None, blk, D), lambda i: (1, i, 0)),
            full((D, D)), full((1, D)),
        ],
        out_specs=pl.BlockSpec((blk, D), lambda i: (i, 0)),
        out_shape=jax.ShapeDtypeStruct((N, D), _f32),
    )(O3, partials, partials, W_o2, b_o2.reshape(1, D))


@jax.jit
def kernel(input_hidden, hidden, last_hidden, batch_assignment, edge_index,
           W_m1, b_m1, W_m2, b_m2, W_mlp1, b_mlp1, W_mlp2, b_mlp2,
           W_o1, b_o1, W_o2, b_o2):
    del batch_assignment
    src = edge_index[0].astype(jnp.int32)
    dst = edge_index[1].astype(jnp.int32)
    src_flat = jnp.zeros((E_PAD,), jnp.int32).at[:E].set(src)
    dstg_flat = jnp.zeros((E_PAD,), jnp.int32).at[:E].set(dst)
    dsts_flat = jnp.full((E_PAD,), TRASH, jnp.int32).at[:E].set(dst)
    srcg = src_flat
    dstg = dstg_flat
    dstg_s = dsts_flat.reshape(NW, NCHUNK_S, CH_S)

    zrows = jnp.zeros((ROWS_PER_TILE, D), _f32)

    P1, P2, O3 = _node_proj(input_hidden, hidden, last_hidden,
                            W_m1, b_m1, W_m2, b_m2, W_o1, b_o1)
    pre = _edge_gather(P1, P2, srcg, dstg)
    msg = _edge_mlp(pre, W_mlp1, b_mlp1, W_mlp2, b_mlp2)
    partials = _edge_scatter(msg, dstg_s, zrows)
    return _combine(O3, partials, W_o2, b_o2)
